# initial kernel scaffold (unmeasured)
import jax
import jax.numpy as jnp
from jax import lax
from jax.experimental import pallas as pl
from jax.experimental.pallas import tpu as pltpu

N_DEV = 8
B, SQ, D = 4, 256, 1024
H_LOC, DH = 8, 128
SCALE = 0.08838834764831843
BF16 = jnp.bfloat16


def kernel(x, Wq, Wo, Wk, Wv):
    def body(x_ref, wq_ref, wo_ref, wk_ref, wv_ref, out_ref,
             q_ref, k_ref, v_ref, attn_ref, comm_ref,
             send_sems, recv_sems):
        my = lax.axis_index("i")
        left = lax.rem(my + N_DEV - 1, N_DEV)
        right = lax.rem(my + 1, N_DEV)

        barrier_sem = pltpu.get_barrier_semaphore()
        for nbr in (left, right):
            pl.semaphore_signal(
                barrier_sem, inc=1,
                device_id=(nbr,), device_id_type=pl.DeviceIdType.MESH,
            )
        pl.semaphore_wait(barrier_sem, 2)

        xb = x_ref[...].reshape(B * SQ, D).astype(BF16)
        q_ref[...] = jnp.dot(xb, wq_ref[...].astype(BF16),
                             preferred_element_type=jnp.float32).astype(BF16)
        k_ref[...] = jnp.dot(xb, wk_ref[...].astype(BF16),
                             preferred_element_type=jnp.float32).astype(BF16)
        v_ref[...] = jnp.dot(xb, wv_ref[...].astype(BF16),
                             preferred_element_type=jnp.float32).astype(BF16)

        for b in range(B):
            rows = pl.ds(b * SQ, SQ)
            for h in range(H_LOC):
                cols = pl.ds(h * DH, DH)
                qs = q_ref[rows, cols]
                ks = k_ref[rows, cols]
                vs = v_ref[rows, cols]
                s = lax.dot_general(
                    qs, ks, (((1,), (1,)), ((), ())),
                    preferred_element_type=jnp.float32,
                ) * SCALE
                m = jnp.max(s, axis=1, keepdims=True)
                p = jnp.exp(s - m)
                l = jnp.sum(p, axis=1, keepdims=True)
                pb = (p / l).astype(BF16)
                attn_ref[rows, cols] = jnp.dot(
                    pb, vs, preferred_element_type=jnp.float32
                ).astype(BF16)

        partial = jnp.dot(attn_ref[...], wo_ref[...].astype(BF16),
                          preferred_element_type=jnp.float32)
        comm_ref[0] = partial.astype(BF16)
        acc = partial

        for h in range(N_DEV - 1):
            rdma = pltpu.make_async_remote_copy(
                src_ref=comm_ref.at[h],
                dst_ref=comm_ref.at[h + 1],
                send_sem=send_sems.at[h],
                recv_sem=recv_sems.at[h + 1],
                device_id=(right,),
                device_id_type=pl.DeviceIdType.MESH,
            )
            rdma.start()
            rdma.wait()
            acc = acc + comm_ref[h + 1].astype(jnp.float32)

        out_ref[...] = acc.reshape(B, SQ, D)

    return pl.pallas_call(
        body,
        out_shape=jax.ShapeDtypeStruct((B, SQ, D), jnp.float32),
        in_specs=[pl.BlockSpec(memory_space=pltpu.VMEM)] * 5,
        out_specs=pl.BlockSpec(memory_space=pltpu.VMEM),
        scratch_shapes=[
            pltpu.VMEM((B * SQ, D), BF16),
            pltpu.VMEM((B * SQ, D), BF16),
            pltpu.VMEM((B * SQ, D), BF16),
            pltpu.VMEM((B * SQ, D), BF16),
            pltpu.VMEM((N_DEV, B * SQ, D), BF16),
            pltpu.SemaphoreType.DMA((N_DEV,)),
            pltpu.SemaphoreType.DMA((N_DEV,)),
        ],
        compiler_params=pltpu.CompilerParams(collective_id=0),
    )(x, Wq, Wo, Wk, Wv)


# baseline (device time: 203487 ns/iter reference)
import jax
import jax.numpy as jnp
from jax import lax
from jax.experimental import pallas as pl
from jax.experimental.pallas import tpu as pltpu

N_DEV = 8
B, SQ, D = 4, 256, 1024
H_LOC, DH = 8, 128
SCALE = 0.08838834764831843
BF16 = jnp.bfloat16


def kernel(x, Wq, Wo, Wk, Wv):
    def body(x_ref, wq_ref, wo_ref, wk_ref, wv_ref, out_ref,
             q_ref, k_ref, v_ref, attn_ref, comm_ref,
             send_sems, recv_sems):
        my = lax.axis_index("i")
        left = lax.rem(my + N_DEV - 1, N_DEV)
        right = lax.rem(my + 1, N_DEV)

        barrier_sem = pltpu.get_barrier_semaphore()
        for nbr in (left, right):
            pl.semaphore_signal(
                barrier_sem, inc=1,
                device_id=(nbr,), device_id_type=pl.DeviceIdType.MESH,
            )
        pl.semaphore_wait(barrier_sem, 2)

        xb = x_ref[...].reshape(B * SQ, D).astype(BF16)
        q_ref[...] = jnp.dot(xb, wq_ref[...].astype(BF16),
                             preferred_element_type=jnp.float32).astype(BF16)
        k_ref[...] = jnp.dot(xb, wk_ref[...].astype(BF16),
                             preferred_element_type=jnp.float32).astype(BF16)
        v_ref[...] = jnp.dot(xb, wv_ref[...].astype(BF16),
                             preferred_element_type=jnp.float32).astype(BF16)

        for b in range(B):
            rows = pl.ds(b * SQ, SQ)
            for h in range(H_LOC):
                cols = pl.ds(h * DH, DH)
                qs = q_ref[rows, cols]
                ks = k_ref[rows, cols]
                vs = v_ref[rows, cols]
                s = lax.dot_general(
                    qs, ks, (((1,), (1,)), ((), ())),
                    preferred_element_type=jnp.float32,
                ) * SCALE
                m = jnp.max(s, axis=1, keepdims=True)
                p = jnp.exp(s - m)
                l = jnp.sum(p, axis=1, keepdims=True)
                pb = (p / l).astype(BF16)
                attn_ref[rows, cols] = jnp.dot(
                    pb, vs, preferred_element_type=jnp.float32
                ).astype(BF16)

        partial = jnp.dot(attn_ref[...], wo_ref[...].astype(BF16),
                          preferred_element_type=jnp.float32)
        comm_ref[0] = partial.astype(BF16)
        acc = partial

        for h in range(N_DEV - 1):
            rdma = pltpu.make_async_remote_copy(
                src_ref=comm_ref.at[h],
                dst_ref=comm_ref.at[h + 1],
                send_sem=send_sems.at[h],
                recv_sem=recv_sems.at[h + 1],
                device_id=(right,),
                device_id_type=pl.DeviceIdType.MESH,
            )
            rdma.start()
            rdma.wait()
            acc = acc + comm_ref[h + 1].astype(jnp.float32)

        out_ref[...] = acc.reshape(B, SQ, D)

    return pl.pallas_call(
        body,
        out_shape=jax.ShapeDtypeStruct((B, SQ, D), jnp.float32),
        in_specs=[pl.BlockSpec(memory_space=pltpu.VMEM)] * 5,
        out_specs=pl.BlockSpec(memory_space=pltpu.VMEM),
        scratch_shapes=[
            pltpu.VMEM((B * SQ, D), BF16),
            pltpu.VMEM((B * SQ, D), BF16),
            pltpu.VMEM((B * SQ, D), BF16),
            pltpu.VMEM((B * SQ, D), BF16),
            pltpu.VMEM((N_DEV, B * SQ, D), BF16),
            pltpu.SemaphoreType.DMA((N_DEV,)),
            pltpu.SemaphoreType.DMA((N_DEV,)),
        ],
        compiler_params=pltpu.CompilerParams(
            collective_id=0,
            vmem_limit_bytes=100 * 1024 * 1024,
        ),
    )(x, Wq, Wo, Wk, Wv)


# device time: 79815 ns/iter; 2.5495x vs baseline; 2.5495x over previous
import jax
import jax.numpy as jnp
from jax import lax
from jax.experimental import pallas as pl
from jax.experimental.pallas import tpu as pltpu

N_DEV = 8
B, SQ, D = 4, 256, 1024
R = B * SQ
CH = R // N_DEV
H_LOC, DH = 8, 128
SCALE = 0.08838834764831843
BF16 = jnp.bfloat16


def kernel(x, Wq, Wo, Wk, Wv):
    def body(x_ref, wq_ref, wo_ref, wk_ref, wv_ref, out_ref,
             q_ref, k_ref, v_ref, attn_ref, w_ref,
             stage_ref, recv_ref,
             send_sems, recv_sems):
        my = lax.axis_index("i")
        b0 = lax.rem(my, 2)
        b1 = lax.rem(my // 2, 2)
        b2 = my // 4

        barrier_sem = pltpu.get_barrier_semaphore()
        for stride, bit in ((1, b0), (2, b1), (4, b2)):
            partner = my + (1 - 2 * bit) * stride
            pl.semaphore_signal(
                barrier_sem, inc=1,
                device_id=(partner,), device_id_type=pl.DeviceIdType.MESH,
            )
        pl.semaphore_wait(barrier_sem, 3)

        xb = x_ref[...].reshape(R, D).astype(BF16)
        q_ref[...] = jnp.dot(xb, wq_ref[...].astype(BF16),
                             preferred_element_type=jnp.float32).astype(BF16)
        k_ref[...] = jnp.dot(xb, wk_ref[...].astype(BF16),
                             preferred_element_type=jnp.float32).astype(BF16)
        v_ref[...] = jnp.dot(xb, wv_ref[...].astype(BF16),
                             preferred_element_type=jnp.float32).astype(BF16)

        for b in range(B):
            rows = pl.ds(b * SQ, SQ)
            for h in range(H_LOC):
                cols = pl.ds(h * DH, DH)
                qs = q_ref[rows, cols]
                ks = k_ref[rows, cols]
                vs = v_ref[rows, cols]
                s = lax.dot_general(
                    qs, ks, (((1,), (1,)), ((), ())),
                    preferred_element_type=jnp.float32,
                ) * SCALE
                m = jnp.max(s, axis=1, keepdims=True)
                p = jnp.exp(s - m)
                l = jnp.sum(p, axis=1, keepdims=True)
                pb = (p / l).astype(BF16)
                attn_ref[rows, cols] = jnp.dot(
                    pb, vs, preferred_element_type=jnp.float32
                ).astype(BF16)

        w_ref[...] = jnp.dot(attn_ref[...], wo_ref[...].astype(BF16),
                             preferred_element_type=jnp.float32)

        def exchange(k, size, src_rows, dst_slot):
            stage_ref[k, pl.ds(0, size), :] = src_rows.astype(BF16)
            rdma = pltpu.make_async_remote_copy(
                src_ref=stage_ref.at[k, pl.ds(0, size)],
                dst_ref=recv_ref.at[k, pl.ds(0, size)],
                send_sem=send_sems.at[k],
                recv_sem=recv_sems.at[k],
                device_id=(dst_slot,),
                device_id_type=pl.DeviceIdType.MESH,
            )
            rdma.start()
            rdma.wait()
            return recv_ref[k, pl.ds(0, size), :]

        pre = my * 0
        for k, (stride, bit) in enumerate(((4, b2), (2, b1), (1, b0))):
            size = CH * stride
            partner = my + (1 - 2 * bit) * stride
            keep = pre + bit * size
            send = pre + (1 - bit) * size
            got = exchange(k, size, w_ref[pl.ds(send, size), :], partner)
            w_ref[pl.ds(keep, size), :] = (
                w_ref[pl.ds(keep, size), :] + got.astype(jnp.float32)
            )
            pre = keep

        out_ref[pl.ds(my * CH, CH), :] = w_ref[pl.ds(my * CH, CH), :]

        for k, stride in enumerate((1, 2, 4)):
            size = CH * stride
            grp = my // stride
            bit = lax.rem(grp, 2)
            partner = my + (1 - 2 * bit) * stride
            mine = grp * size
            theirs = (grp + (1 - 2 * bit)) * size
            got = exchange(3 + k, size, out_ref[pl.ds(mine, size), :], partner)
            out_ref[pl.ds(theirs, size), :] = got.astype(jnp.float32)

    out = pl.pallas_call(
        body,
        out_shape=jax.ShapeDtypeStruct((R, D), jnp.float32),
        in_specs=[pl.BlockSpec(memory_space=pltpu.VMEM)] * 5,
        out_specs=pl.BlockSpec(memory_space=pltpu.VMEM),
        scratch_shapes=[
            pltpu.VMEM((R, D), BF16),
            pltpu.VMEM((R, D), BF16),
            pltpu.VMEM((R, D), BF16),
            pltpu.VMEM((R, D), BF16),
            pltpu.VMEM((R, D), jnp.float32),
            pltpu.VMEM((6, R // 2, D), BF16),
            pltpu.VMEM((6, R // 2, D), BF16),
            pltpu.SemaphoreType.DMA((6,)),
            pltpu.SemaphoreType.DMA((6,)),
        ],
        compiler_params=pltpu.CompilerParams(
            collective_id=0,
            vmem_limit_bytes=100 * 1024 * 1024,
        ),
    )(x, Wq, Wo, Wk, Wv)
    return out.reshape(B, SQ, D)


# device time: 54799 ns/iter; 3.7133x vs baseline; 1.4565x over previous
import jax
import jax.numpy as jnp
from jax import lax
from jax.experimental import pallas as pl
from jax.experimental.pallas import tpu as pltpu

N_DEV = 8
B, SQ, D = 4, 256, 1024
R = B * SQ
CH = R // N_DEV
H_LOC, DH = 8, 128
SCALE = 0.08838834764831843
BF16 = jnp.bfloat16
MESH = pl.DeviceIdType.MESH


def kernel(x, Wq, Wo, Wk, Wv):
    def body(x_ref, wq_ref, wo_ref, wk_ref, wv_ref, out_ref,
             wqb_ref, wkb_ref, wvb_ref, wob_ref,
             q_ref, k_ref, v_ref, attn_ref, red_ref,
             stage_ref, rsrecv_ref,
             rs_send_sems, rs_recv_sems, ag_send_sems, ag_recv_sems):
        my = lax.axis_index("i")

        barrier_sem = pltpu.get_barrier_semaphore()
        for t in range(N_DEV):
            @pl.when(my != t)
            def _():
                pl.semaphore_signal(barrier_sem, inc=1,
                                    device_id=(t,), device_id_type=MESH)
        pl.semaphore_wait(barrier_sem, N_DEV - 1)

        wqb_ref[...] = wq_ref[...].astype(BF16)
        wkb_ref[...] = wk_ref[...].astype(BF16)
        wvb_ref[...] = wv_ref[...].astype(BF16)
        wob_ref[...] = wo_ref[...].astype(BF16)

        def rs_desc(src_slot, dst_slot, sem_slot):
            return pltpu.make_async_remote_copy(
                src_ref=stage_ref.at[src_slot],
                dst_ref=rsrecv_ref.at[dst_slot],
                send_sem=rs_send_sems.at[src_slot],
                recv_sem=rs_recv_sems.at[sem_slot],
                device_id=(src_slot,), device_id_type=MESH,
            )

        for b in range(B):
            xb = x_ref[b].astype(BF16)
            q_ref[...] = jnp.dot(xb, wqb_ref[...],
                                 preferred_element_type=jnp.float32).astype(BF16)
            k_ref[...] = jnp.dot(xb, wkb_ref[...],
                                 preferred_element_type=jnp.float32).astype(BF16)
            v_ref[...] = jnp.dot(xb, wvb_ref[...],
                                 preferred_element_type=jnp.float32).astype(BF16)

            for h in range(H_LOC):
                cols = pl.ds(h * DH, DH)
                qs = q_ref[:, cols]
                ks = k_ref[:, cols]
                vs = v_ref[:, cols]
                s = lax.dot_general(
                    qs, ks, (((1,), (1,)), ((), ())),
                    preferred_element_type=jnp.float32,
                ) * SCALE
                m = jnp.max(s, axis=1, keepdims=True)
                p = jnp.exp(s - m)
                l = jnp.sum(p, axis=1, keepdims=True)
                pb = (p / l).astype(BF16)
                attn_ref[:, cols] = jnp.dot(
                    pb, vs, preferred_element_type=jnp.float32
                ).astype(BF16)

            partial = jnp.dot(attn_ref[...], wob_ref[...],
                              preferred_element_type=jnp.float32)

            for oc in range(2):
                c = 2 * b + oc
                rows = partial[oc * CH:(oc + 1) * CH, :]

                @pl.when(my == c)
                def _(rows=rows):
                    red_ref[...] = rows

                @pl.when(my != c)
                def _(rows=rows, c=c):
                    stage_ref[c] = rows.astype(BF16)
                    rs_desc(c, my, my).start()

        for c in range(N_DEV):
            @pl.when(my != c)
            def _(c=c):
                rs_desc(c, c, c).wait_send()
        for s_id in range(N_DEV):
            @pl.when(my == s_id)
            def _(s_id=s_id):
                rsrecv_ref[s_id] = jnp.zeros((CH, D), BF16)

            @pl.when(my != s_id)
            def _(s_id=s_id):
                rs_desc(s_id, s_id, s_id).wait_recv()

        red = red_ref[...]
        for s_id in range(N_DEV):
            red = red + rsrecv_ref[s_id].astype(jnp.float32)
        out_ref[pl.ds(my * CH, CH), :] = red.astype(BF16)

        def ag_desc(region, send_slot, recv_slot, target):
            blk = pl.ds(region * CH, CH)
            return pltpu.make_async_remote_copy(
                src_ref=out_ref.at[blk],
                dst_ref=out_ref.at[blk],
                send_sem=ag_send_sems.at[send_slot],
                recv_sem=ag_recv_sems.at[recv_slot],
                device_id=(target,), device_id_type=MESH,
            )

        for t in range(N_DEV):
            @pl.when(my != t)
            def _(t=t):
                ag_desc(my, t, my, t).start()
        for s_id in range(N_DEV):
            @pl.when(my != s_id)
            def _(s_id=s_id):
                ag_desc(s_id, s_id, s_id, s_id).wait_recv()
        for t in range(N_DEV):
            @pl.when(my != t)
            def _(t=t):
                ag_desc(my, t, t, t).wait_send()

    out = pl.pallas_call(
        body,
        out_shape=jax.ShapeDtypeStruct((R, D), BF16),
        in_specs=[pl.BlockSpec(memory_space=pltpu.VMEM)] * 5,
        out_specs=pl.BlockSpec(memory_space=pltpu.VMEM),
        scratch_shapes=[
            pltpu.VMEM((D, D), BF16),
            pltpu.VMEM((D, D), BF16),
            pltpu.VMEM((D, D), BF16),
            pltpu.VMEM((D, D), BF16),
            pltpu.VMEM((SQ, D), BF16),
            pltpu.VMEM((SQ, D), BF16),
            pltpu.VMEM((SQ, D), BF16),
            pltpu.VMEM((SQ, D), BF16),
            pltpu.VMEM((CH, D), jnp.float32),
            pltpu.VMEM((N_DEV, CH, D), BF16),
            pltpu.VMEM((N_DEV, CH, D), BF16),
            pltpu.SemaphoreType.DMA((N_DEV,)),
            pltpu.SemaphoreType.DMA((N_DEV,)),
            pltpu.SemaphoreType.DMA((N_DEV,)),
            pltpu.SemaphoreType.DMA((N_DEV,)),
        ],
        compiler_params=pltpu.CompilerParams(
            collective_id=0,
            vmem_limit_bytes=100 * 1024 * 1024,
        ),
    )(x, Wq, Wo, Wk, Wv)
    return out.reshape(B, SQ, D)
